# initial kernel scaffold (unmeasured)
import jax
import jax.numpy as jnp
from jax import lax
from jax.experimental import pallas as pl
from jax.experimental.pallas import tpu as pltpu


def kernel(
    x,
):
    def body(*refs):
        pass

    out_shape = jax.ShapeDtypeStruct(..., jnp.float32)
    return pl.pallas_call(body, out_shape=out_shape)(...)


# baseline (device time: 300536 ns/iter reference)
import jax
import jax.numpy as jnp
from jax import lax
from jax.experimental import pallas as pl
from jax.experimental.pallas import tpu as pltpu

N_DEV = 32


def kernel(x):
    m, n = x.shape
    chunk = m // N_DEV

    def body(x_ref, out_ref, comm_ref, send_sems, recv_sems):
        my = lax.axis_index("i")
        left = jnp.mod(my - 1, N_DEV)
        right = jnp.mod(my + 1, N_DEV)

        barrier_sem = pltpu.get_barrier_semaphore()
        for nbr in (left, right):
            pl.semaphore_signal(
                barrier_sem, inc=1,
                device_id=(nbr,), device_id_type=pl.DeviceIdType.MESH,
            )
        pl.semaphore_wait(barrier_sem, 2)

        def rows(c):
            return pl.ds(c * chunk, chunk)

        comm_ref[0, :, :] = x_ref[rows(my), :]

        for h in range(2 * N_DEV - 2):
            s = h % 2
            r = (h + 1) % 2
            rdma = pltpu.make_async_remote_copy(
                src_ref=comm_ref.at[s],
                dst_ref=comm_ref.at[r],
                send_sem=send_sems.at[s],
                recv_sem=recv_sems.at[r],
                device_id=(right,),
                device_id_type=pl.DeviceIdType.MESH,
            )
            rdma.start()
            rdma.wait()

            if h < N_DEV - 2:
                c = jnp.mod(my - h - 1, N_DEV)
                comm_ref[r, :, :] = comm_ref[r, :, :] + x_ref[rows(c), :]
            elif h == N_DEV - 2:
                c = jnp.mod(my + 1, N_DEV)
                comm_ref[r, :, :] = comm_ref[r, :, :] + x_ref[rows(c), :]
                out_ref[rows(c), :] = comm_ref[r, :, :]
            else:
                g = h - (N_DEV - 1)
                c = jnp.mod(my - g, N_DEV)
                out_ref[rows(c), :] = comm_ref[r, :, :]

    return pl.pallas_call(
        body,
        out_shape=jax.ShapeDtypeStruct((m, n), x.dtype),
        in_specs=[pl.BlockSpec(memory_space=pltpu.VMEM)],
        out_specs=pl.BlockSpec(memory_space=pltpu.VMEM),
        scratch_shapes=[
            pltpu.VMEM((2, chunk, n), x.dtype),
            pltpu.SemaphoreType.DMA((2,)),
            pltpu.SemaphoreType.DMA((2,)),
        ],
        compiler_params=pltpu.CompilerParams(collective_id=0),
    )(x)


# device time: 239611 ns/iter; 1.2543x vs baseline; 1.2543x over previous
import jax
import jax.numpy as jnp
from jax import lax
from jax.experimental import pallas as pl
from jax.experimental.pallas import tpu as pltpu

N_DEV = 32
N_RING = 4
SIGMA = (1, 1, -1, -1)
OFF = (0, 16, 32, 48)
SUB = 16


def kernel(x):
    m, n = x.shape
    chunk = m // N_DEV

    def body(x_ref, out_ref, comm_ref, send_sems, recv_sems):
        my = lax.axis_index("i")
        left = jnp.mod(my - 1, N_DEV)
        right = jnp.mod(my + 1, N_DEV)

        barrier_sem = pltpu.get_barrier_semaphore()
        for nbr in (left, right):
            pl.semaphore_signal(
                barrier_sem, inc=1,
                device_id=(nbr,), device_id_type=pl.DeviceIdType.MESH,
            )
        pl.semaphore_wait(barrier_sem, 2)

        def rows(c, j):
            return pl.ds(c * chunk + OFF[j], SUB)

        def mk(j, h):
            return pltpu.make_async_remote_copy(
                src_ref=comm_ref.at[j, h % 2],
                dst_ref=comm_ref.at[j, (h + 1) % 2],
                send_sem=send_sems.at[j, h % 2],
                recv_sem=recv_sems.at[j, (h + 1) % 2],
                device_id=(right if SIGMA[j] > 0 else left,),
                device_id_type=pl.DeviceIdType.MESH,
            )

        rd = []
        for j in range(N_RING):
            comm_ref[j, 0, :, :] = x_ref[rows(my, j), :]
        for j in range(N_RING):
            d = mk(j, 0)
            d.start()
            rd.append(d)

        for h in range(1, 2 * N_DEV - 1):
            for j in range(N_RING):
                s = SIGMA[j]
                rd[j].wait()
                hp = h - 1
                r = h % 2
                if hp < N_DEV - 2:
                    c = jnp.mod(my - s * (hp + 1), N_DEV)
                    comm_ref[j, r, :, :] = comm_ref[j, r, :, :] + x_ref[rows(c, j), :]
                elif hp == N_DEV - 2:
                    c = jnp.mod(my + s, N_DEV)
                    comm_ref[j, r, :, :] = comm_ref[j, r, :, :] + x_ref[rows(c, j), :]
                if h < 2 * N_DEV - 2:
                    rd[j] = mk(j, h)
                    rd[j].start()
                if hp == N_DEV - 2:
                    c = jnp.mod(my + s, N_DEV)
                    out_ref[rows(c, j), :] = comm_ref[j, r, :, :]
                elif hp > N_DEV - 2:
                    c = jnp.mod(my - s * (hp - (N_DEV - 1)), N_DEV)
                    out_ref[rows(c, j), :] = comm_ref[j, r, :, :]

    return pl.pallas_call(
        body,
        out_shape=jax.ShapeDtypeStruct((m, n), x.dtype),
        in_specs=[pl.BlockSpec(memory_space=pltpu.VMEM)],
        out_specs=pl.BlockSpec(memory_space=pltpu.VMEM),
        scratch_shapes=[
            pltpu.VMEM((N_RING, 2, SUB, n), x.dtype),
            pltpu.SemaphoreType.DMA((N_RING, 2)),
            pltpu.SemaphoreType.DMA((N_RING, 2)),
        ],
        compiler_params=pltpu.CompilerParams(collective_id=0),
    )(x)


# device time: 160830 ns/iter; 1.8687x vs baseline; 1.4898x over previous
import jax
import jax.numpy as jnp
from jax import lax
from jax.experimental import pallas as pl
from jax.experimental.pallas import tpu as pltpu

N_DEV = 32
N_RING = 4
SIGMA = (1, 1, -1, -1)
OFF = (0, 16, 32, 48)
SUB = 16


def kernel(x):
    m, n = x.shape
    chunk = m // N_DEV

    def _logical_to_pos(l):
        z = l // 8
        r = l % 8
        y = r // 2
        xp = r % 2
        x = jnp.where(y % 2 == 0, xp, 1 - xp)
        q = z * 4 + jnp.where(z % 2 == 0, y, 3 - y)
        return jnp.where(x == 0, q, 31 - q)

    def _pos_to_logical(p):
        in_x0 = p < 16
        q = jnp.where(in_x0, p, 31 - p)
        z = q // 4
        ym = q % 4
        y = jnp.where(z % 2 == 0, ym, 3 - ym)
        x = jnp.where(in_x0, 0, 1)
        return z * 8 + y * 2 + jnp.where(y % 2 == 0, x, 1 - x)

    def body(x_ref, out_ref, comm_ref, send_sems, recv_sems):
        my = _logical_to_pos(lax.axis_index("i"))
        left = _pos_to_logical(jnp.mod(my - 1, N_DEV))
        right = _pos_to_logical(jnp.mod(my + 1, N_DEV))

        barrier_sem = pltpu.get_barrier_semaphore()
        for nbr in (left, right):
            pl.semaphore_signal(
                barrier_sem, inc=1,
                device_id=(nbr,), device_id_type=pl.DeviceIdType.MESH,
            )
        pl.semaphore_wait(barrier_sem, 2)

        def rows(c, j):
            return pl.ds(c * chunk + OFF[j], SUB)

        def mk(j, h):
            return pltpu.make_async_remote_copy(
                src_ref=comm_ref.at[j, h % 2],
                dst_ref=comm_ref.at[j, (h + 1) % 2],
                send_sem=send_sems.at[j, h % 2],
                recv_sem=recv_sems.at[j, (h + 1) % 2],
                device_id=(right if SIGMA[j] > 0 else left,),
                device_id_type=pl.DeviceIdType.MESH,
            )

        rd = []
        for j in range(N_RING):
            comm_ref[j, 0, :, :] = x_ref[rows(my, j), :]
        for j in range(N_RING):
            d = mk(j, 0)
            d.start()
            rd.append(d)

        for h in range(1, 2 * N_DEV - 1):
            for j in range(N_RING):
                s = SIGMA[j]
                rd[j].wait()
                hp = h - 1
                r = h % 2
                if hp < N_DEV - 2:
                    c = jnp.mod(my - s * (hp + 1), N_DEV)
                    comm_ref[j, r, :, :] = comm_ref[j, r, :, :] + x_ref[rows(c, j), :]
                elif hp == N_DEV - 2:
                    c = jnp.mod(my + s, N_DEV)
                    comm_ref[j, r, :, :] = comm_ref[j, r, :, :] + x_ref[rows(c, j), :]
                if h < 2 * N_DEV - 2:
                    rd[j] = mk(j, h)
                    rd[j].start()
                if hp == N_DEV - 2:
                    c = jnp.mod(my + s, N_DEV)
                    out_ref[rows(c, j), :] = comm_ref[j, r, :, :]
                elif hp > N_DEV - 2:
                    c = jnp.mod(my - s * (hp - (N_DEV - 1)), N_DEV)
                    out_ref[rows(c, j), :] = comm_ref[j, r, :, :]

    return pl.pallas_call(
        body,
        out_shape=jax.ShapeDtypeStruct((m, n), x.dtype),
        in_specs=[pl.BlockSpec(memory_space=pltpu.VMEM)],
        out_specs=pl.BlockSpec(memory_space=pltpu.VMEM),
        scratch_shapes=[
            pltpu.VMEM((N_RING, 2, SUB, n), x.dtype),
            pltpu.SemaphoreType.DMA((N_RING, 2)),
            pltpu.SemaphoreType.DMA((N_RING, 2)),
        ],
        compiler_params=pltpu.CompilerParams(collective_id=0),
    )(x)


# device time: 152704 ns/iter; 1.9681x vs baseline; 1.0532x over previous
import jax
import jax.numpy as jnp
from jax import lax
from jax.experimental import pallas as pl
from jax.experimental.pallas import tpu as pltpu

N_DEV = 32
HALF = N_DEV // 2


def kernel(x):
    m, n = x.shape
    chunk = m // N_DEV

    def _logical_to_pos(l):
        z = l // 8
        r = l % 8
        y = r // 2
        xp = r % 2
        x_ = jnp.where(y % 2 == 0, xp, 1 - xp)
        q = z * 4 + jnp.where(z % 2 == 0, y, 3 - y)
        return jnp.where(x_ == 0, q, 31 - q)

    def _pos_to_logical(p):
        in_x0 = p < 16
        q = jnp.where(in_x0, p, 31 - p)
        z = q // 4
        ym = q % 4
        y = jnp.where(z % 2 == 0, ym, 3 - ym)
        x_ = jnp.where(in_x0, 0, 1)
        return z * 8 + y * 2 + jnp.where(y % 2 == 0, x_, 1 - x_)

    def body(x_ref, out_ref, rbuf, lbuf, rs_sems, rr_sems, ls_sems, lr_sems):
        my = _logical_to_pos(lax.axis_index("i"))
        left = _pos_to_logical(jnp.mod(my - 1, N_DEV))
        right = _pos_to_logical(jnp.mod(my + 1, N_DEV))

        barrier_sem = pltpu.get_barrier_semaphore()
        for nbr in (left, right):
            pl.semaphore_signal(
                barrier_sem, inc=1,
                device_id=(nbr,), device_id_type=pl.DeviceIdType.MESH,
            )
        pl.semaphore_wait(barrier_sem, 2)

        def rows(c):
            return pl.ds(c * chunk, chunk)

        def mk_r(h):
            return pltpu.make_async_remote_copy(
                src_ref=rbuf.at[h % 2],
                dst_ref=rbuf.at[(h + 1) % 2],
                send_sem=rs_sems.at[h % 2],
                recv_sem=rr_sems.at[(h + 1) % 2],
                device_id=(right,),
                device_id_type=pl.DeviceIdType.MESH,
            )

        def mk_l(h):
            return pltpu.make_async_remote_copy(
                src_ref=lbuf.at[h % 2],
                dst_ref=lbuf.at[(h + 1) % 2],
                send_sem=ls_sems.at[h % 2],
                recv_sem=lr_sems.at[(h + 1) % 2],
                device_id=(left,),
                device_id_type=pl.DeviceIdType.MESH,
            )

        rbuf[0, :, :] = x_ref[rows(my), :]
        lbuf[0, :, :] = x_ref[rows(jnp.mod(my + 1, N_DEV)), :]
        rd_r = mk_r(0)
        rd_r.start()
        rd_l = mk_l(0)
        rd_l.start()

        for t in range(1, 2 * HALF + 1):
            hr = t - 1
            rd_r.wait()
            rr = t % 2
            if hr < HALF - 1:
                c = jnp.mod(my - t, N_DEV)
                rbuf[rr, :, :] = rbuf[rr, :, :] + x_ref[rows(c), :]
                rd_r = mk_r(t)
                rd_r.start()
            elif hr == HALF - 1:
                c_own = jnp.mod(my + HALF, N_DEV)
                red = rbuf[rr, :, :] + lbuf[1, :, :] + x_ref[rows(c_own), :]
                rbuf[rr, :, :] = red
                lbuf[1, :, :] = red
                rd_r = mk_r(t)
                rd_r.start()
                out_ref[rows(c_own), :] = red
            else:
                if t <= 2 * HALF - 1:
                    rd_r = mk_r(t)
                    rd_r.start()
                c = jnp.mod(my + 2 * HALF - t, N_DEV)
                out_ref[rows(c), :] = rbuf[rr, :, :]

            if t <= HALF - 1:
                rd_l.wait()
                rl = t % 2
                hl = t - 1
                if hl < HALF - 2:
                    c = jnp.mod(my + 1 + t, N_DEV)
                    lbuf[rl, :, :] = lbuf[rl, :, :] + x_ref[rows(c), :]
                    rd_l = mk_l(t)
                    rd_l.start()
            elif t == HALF:
                rd_l = mk_l(HALF - 1)
                rd_l.start()
            else:
                hl = t - 2
                if hl <= 2 * HALF - 3:
                    rd_l.wait()
                    rl = (hl + 1) % 2
                    if hl + 1 <= 2 * HALF - 3:
                        rd_l = mk_l(hl + 1)
                        rd_l.start()
                    c = jnp.mod(my + 2 + hl, N_DEV)
                    out_ref[rows(c), :] = lbuf[rl, :, :]

    return pl.pallas_call(
        body,
        out_shape=jax.ShapeDtypeStruct((m, n), x.dtype),
        in_specs=[pl.BlockSpec(memory_space=pltpu.VMEM)],
        out_specs=pl.BlockSpec(memory_space=pltpu.VMEM),
        scratch_shapes=[
            pltpu.VMEM((2, chunk, n), x.dtype),
            pltpu.VMEM((2, chunk, n), x.dtype),
            pltpu.SemaphoreType.DMA((2,)),
            pltpu.SemaphoreType.DMA((2,)),
            pltpu.SemaphoreType.DMA((2,)),
            pltpu.SemaphoreType.DMA((2,)),
        ],
        compiler_params=pltpu.CompilerParams(collective_id=0),
    )(x)


# device time: 101788 ns/iter; 2.9526x vs baseline; 1.5002x over previous
import jax
import jax.numpy as jnp
from jax import lax
from jax.experimental import pallas as pl
from jax.experimental.pallas import tpu as pltpu

N_DEV = 32
HALF = N_DEV // 2
SUBS = 4
SUB = 16


def kernel(x):
    m, n = x.shape
    chunk = m // N_DEV

    def _logical_to_pos(l):
        z = l // 8
        r = l % 8
        y = r // 2
        xp = r % 2
        x_ = jnp.where(y % 2 == 0, xp, 1 - xp)
        q = z * 4 + jnp.where(z % 2 == 0, y, 3 - y)
        return jnp.where(x_ == 0, q, 31 - q)

    def _pos_to_logical(p):
        in_x0 = p < 16
        q = jnp.where(in_x0, p, 31 - p)
        z = q // 4
        ym = q % 4
        y = jnp.where(z % 2 == 0, ym, 3 - ym)
        x_ = jnp.where(in_x0, 0, 1)
        return z * 8 + y * 2 + jnp.where(y % 2 == 0, x_, 1 - x_)

    def body(x_ref, out_ref, rbuf, lbuf, rs_sems, rr_sems, ls_sems, lr_sems):
        my = _logical_to_pos(lax.axis_index("i"))
        left = _pos_to_logical(jnp.mod(my - 1, N_DEV))
        right = _pos_to_logical(jnp.mod(my + 1, N_DEV))

        barrier_sem = pltpu.get_barrier_semaphore()
        for nbr in (left, right):
            pl.semaphore_signal(
                barrier_sem, inc=1,
                device_id=(nbr,), device_id_type=pl.DeviceIdType.MESH,
            )
        pl.semaphore_wait(barrier_sem, 2)

        def rows(c, k):
            return pl.ds(c * chunk + k * SUB, SUB)

        def mk_r(k, h):
            return pltpu.make_async_remote_copy(
                src_ref=rbuf.at[k, h % 2],
                dst_ref=rbuf.at[k, (h + 1) % 2],
                send_sem=rs_sems.at[k, h % 2],
                recv_sem=rr_sems.at[k, (h + 1) % 2],
                device_id=(right,),
                device_id_type=pl.DeviceIdType.MESH,
            )

        def mk_l(k, h):
            return pltpu.make_async_remote_copy(
                src_ref=lbuf.at[k, h % 2],
                dst_ref=lbuf.at[k, (h + 1) % 2],
                send_sem=ls_sems.at[k, h % 2],
                recv_sem=lr_sems.at[k, (h + 1) % 2],
                device_id=(left,),
                device_id_type=pl.DeviceIdType.MESH,
            )

        rd_r = [None] * SUBS
        rd_l = [None] * SUBS
        for k in range(SUBS):
            rbuf[k, 0, :, :] = x_ref[rows(my, k), :]
            lbuf[k, 0, :, :] = x_ref[rows(jnp.mod(my + 1, N_DEV), k), :]
        for k in range(SUBS):
            rd_r[k] = mk_r(k, 0)
            rd_r[k].start()
            rd_l[k] = mk_l(k, 0)
            rd_l[k].start()

        def right_step(k, t):
            hr = t - 1
            rd_r[k].wait()
            rr = t % 2
            if hr < HALF - 1:
                c = jnp.mod(my - t, N_DEV)
                rbuf[k, rr, :, :] = rbuf[k, rr, :, :] + x_ref[rows(c, k), :]
                rd_r[k] = mk_r(k, t)
                rd_r[k].start()
            elif hr == HALF - 1:
                c_own = jnp.mod(my + HALF, N_DEV)
                red = (
                    rbuf[k, rr, :, :]
                    + lbuf[k, 1, :, :]
                    + x_ref[rows(c_own, k), :]
                )
                rbuf[k, rr, :, :] = red
                lbuf[k, 1, :, :] = red
                rd_r[k] = mk_r(k, t)
                rd_r[k].start()
                out_ref[rows(c_own, k), :] = red
            else:
                if t <= 2 * HALF - 1:
                    rd_r[k] = mk_r(k, t)
                    rd_r[k].start()
                c = jnp.mod(my + 2 * HALF - t, N_DEV)
                out_ref[rows(c, k), :] = rbuf[k, rr, :, :]

        def left_step(k, t):
            if t <= HALF - 1:
                rd_l[k].wait()
                rl = t % 2
                hl = t - 1
                if hl < HALF - 2:
                    c = jnp.mod(my + 1 + t, N_DEV)
                    lbuf[k, rl, :, :] = (
                        lbuf[k, rl, :, :] + x_ref[rows(c, k), :]
                    )
                    rd_l[k] = mk_l(k, t)
                    rd_l[k].start()
            elif t == HALF:
                rd_l[k] = mk_l(k, HALF - 1)
                rd_l[k].start()
            else:
                hl = t - 2
                if hl <= 2 * HALF - 3:
                    rd_l[k].wait()
                    rl = (hl + 1) % 2
                    if hl + 1 <= 2 * HALF - 3:
                        rd_l[k] = mk_l(k, hl + 1)
                        rd_l[k].start()
                    c = jnp.mod(my + 2 + hl, N_DEV)
                    out_ref[rows(c, k), :] = lbuf[k, rl, :, :]

        for t in range(1, 2 * HALF + 1):
            for k in range(SUBS):
                right_step(k, t)
                left_step(k, t)

    return pl.pallas_call(
        body,
        out_shape=jax.ShapeDtypeStruct((m, n), x.dtype),
        in_specs=[pl.BlockSpec(memory_space=pltpu.VMEM)],
        out_specs=pl.BlockSpec(memory_space=pltpu.VMEM),
        scratch_shapes=[
            pltpu.VMEM((SUBS, 2, SUB, n), x.dtype),
            pltpu.VMEM((SUBS, 2, SUB, n), x.dtype),
            pltpu.SemaphoreType.DMA((SUBS, 2)),
            pltpu.SemaphoreType.DMA((SUBS, 2)),
            pltpu.SemaphoreType.DMA((SUBS, 2)),
            pltpu.SemaphoreType.DMA((SUBS, 2)),
        ],
        compiler_params=pltpu.CompilerParams(collective_id=0),
    )(x)
